# baseline (speedup 1.0000x reference)
import functools

import jax
import jax.numpy as jnp
from jax import lax
from jax.experimental import pallas as pl
from jax.experimental.pallas import tpu as pltpu
from jax.experimental.pallas import tpu_sc as plsc

N = 10000
E = 320000
D = 128
H = 64
C = 40
G = 10

NC, NS = 2, 16
NW = NC * NS
CH = 128

NPAD = 10240
ROWS_PER_TILE = NPAD // NS
PAD_SPREAD = NPAD - N

NBUF = 5
MSG_STEPS = 80
EPT = MSG_STEPS * CH
EPAD = EPT * NW
STEADY = MSG_STEPS - NBUF

DEG_SLICE = NPAD // NS

_MESH = plsc.VectorSubcoreMesh(core_axis_name="c", subcore_axis_name="s")
_SC_PARAMS = pltpu.CompilerParams(use_tc_tiling_on_sc=False)


@functools.partial(
    pl.kernel,
    out_type=jax.ShapeDtypeStruct((NC, 2, NPAD), jnp.float32),
    mesh=_MESH,
    scratch_types=[
        pltpu.VMEM((MSG_STEPS, CH), jnp.int32),
        pltpu.VMEM((MSG_STEPS, CH), jnp.int32),
        pltpu.VMEM((CH,), jnp.float32),
        pltpu.VMEM((DEG_SLICE,), jnp.float32),
        pltpu.VMEM_SHARED((NPAD,), jnp.float32),
        pltpu.VMEM_SHARED((NPAD,), jnp.float32),
    ],
    compiler_params=_SC_PARAMS,
)
def _deg_kernel(src_hbm, dst_hbm, out_hbm, srcs, dsts, ones_v, zer_v,
                acc_s, acc_d):
    cid = lax.axis_index("c")
    sid = lax.axis_index("s")
    wid = cid * NS + sid

    @pl.loop(0, DEG_SLICE // 16)
    def _zero(i):
        zer_v[pl.ds(i * 16, 16)] = jnp.zeros((16,), jnp.float32)

    @pl.loop(0, CH // 16)
    def _ones(i):
        ones_v[pl.ds(i * 16, 16)] = jnp.ones((16,), jnp.float32)

    pltpu.sync_copy(zer_v, acc_s.at[pl.ds(sid * DEG_SLICE, DEG_SLICE)])
    pltpu.sync_copy(zer_v, acc_d.at[pl.ds(sid * DEG_SLICE, DEG_SLICE)])
    pltpu.sync_copy(src_hbm.at[wid], srcs)
    pltpu.sync_copy(dst_hbm.at[wid], dsts)
    plsc.subcore_barrier()

    @pl.loop(0, MSG_STEPS)
    def _step(j):
        pltpu.sync_copy(ones_v, acc_s.at[srcs.at[j]], add=True)
        pltpu.sync_copy(ones_v, acc_d.at[dsts.at[j]], add=True)

    plsc.subcore_barrier()
    pltpu.sync_copy(acc_s.at[pl.ds(sid * DEG_SLICE, DEG_SLICE)],
                    out_hbm.at[cid, 0, pl.ds(sid * DEG_SLICE, DEG_SLICE)])
    pltpu.sync_copy(acc_d.at[pl.ds(sid * DEG_SLICE, DEG_SLICE)],
                    out_hbm.at[cid, 1, pl.ds(sid * DEG_SLICE, DEG_SLICE)])


_MSG_SCRATCH = (
    [pltpu.VMEM((MSG_STEPS, CH), jnp.int32)] * 2 +
    [pltpu.VMEM((CH, H), jnp.float32)] * (NBUF + 1) +
    [pltpu.SemaphoreType.DMA] * NBUF +
    [pltpu.VMEM_SHARED((NPAD, H), jnp.float32)]
)


@functools.partial(
    pl.kernel,
    out_type=jax.ShapeDtypeStruct((NC, NPAD, H), jnp.float32),
    mesh=_MESH,
    scratch_types=_MSG_SCRATCH,
    compiler_params=_SC_PARAMS,
)
def _msg_kernel(src_hbm, dst_hbm, feat_hbm, out_hbm, srcs, dsts, *rest):
    rowbufs = rest[:NBUF]
    zbuf = rest[NBUF]
    sems = rest[NBUF + 1:2 * NBUF + 1]
    acc = rest[2 * NBUF + 1]
    cid = lax.axis_index("c")
    sid = lax.axis_index("s")
    wid = cid * NS + sid

    @pl.loop(0, CH)
    def _zrow(i):
        @pl.loop(0, H // 16)
        def _zcol(j):
            zbuf[i, pl.ds(j * 16, 16)] = jnp.zeros((16,), jnp.float32)

    for i in range(ROWS_PER_TILE // CH):
        pltpu.sync_copy(zbuf, acc.at[pl.ds(sid * ROWS_PER_TILE + i * CH, CH)])
    plsc.subcore_barrier()

    pltpu.sync_copy(src_hbm.at[wid], srcs)
    pltpu.sync_copy(dst_hbm.at[wid], dsts)

    for b in range(NBUF):
        pltpu.async_copy(feat_hbm.at[srcs.at[b]], rowbufs[b], sems[b])

    @pl.loop(0, STEADY, step=NBUF)
    def _step(j):
        for b in range(NBUF):
            rows, sem = rowbufs[b], sems[b]
            pltpu.make_async_copy(feat_hbm.at[srcs.at[j + b]], rows,
                                  sem).wait()
            pltpu.sync_copy(rows, acc.at[dsts.at[j + b]], add=True)
            pltpu.async_copy(feat_hbm.at[srcs.at[j + b + NBUF]], rows, sem)

    for b in range(NBUF):
        rows, sem = rowbufs[b], sems[b]
        c = STEADY + b
        pltpu.make_async_copy(feat_hbm.at[srcs.at[c]], rows, sem).wait()
        pltpu.sync_copy(rows, acc.at[dsts.at[c]], add=True)

    plsc.subcore_barrier()
    for i in range(ROWS_PER_TILE // CH):
        r0 = sid * ROWS_PER_TILE + i * CH
        pltpu.sync_copy(acc.at[pl.ds(r0, CH)], out_hbm.at[cid, pl.ds(r0, CH)])


_PREC = lax.Precision.HIGHEST


def _gelu(x):
    return 0.5 * x * (1.0 + lax.erf(x * 0.7071067811865476))


def _rdeg(degp_ref, which):
    d = degp_ref[0, which, :N] + degp_ref[1, which, :N]
    return lax.rsqrt(jnp.maximum(d, 1.0)).reshape(N, 1)


def _tc_a_body(x_ref, w0_ref, degp_ref, y_ref):
    y0 = jnp.dot(x_ref[...], w0_ref[...], precision=_PREC)
    y_ref[:N] = y0 * _rdeg(degp_ref, 0)
    y_ref[N:] = jnp.zeros((NPAD - N, H), jnp.float32)


def _tc_b_body(p_ref, degp_ref, b0_ref, w1_ref, z_ref):
    m = p_ref[0, :N] + p_ref[1, :N]
    h = _gelu(m * _rdeg(degp_ref, 1) + b0_ref[...])
    z_ref[:N] = jnp.dot(h, w1_ref[...], precision=_PREC) * _rdeg(degp_ref, 0)
    z_ref[N:] = jnp.zeros((NPAD - N, H), jnp.float32)


def _tc_c_body(p10_ref, din_ref, b1_ref, wl_ref, bl_ref, o_ref):
    m = p10_ref[0] + p10_ref[1]
    rin = lax.rsqrt(jnp.maximum(din_ref[...], 1.0))
    h = _gelu(m * rin + b1_ref[...])
    o_ref[...] = jnp.dot(h, wl_ref[...], precision=_PREC) + bl_ref[...]


_tc_a = pl.pallas_call(
    _tc_a_body,
    out_shape=jax.ShapeDtypeStruct((NPAD, H), jnp.float32),
)

_tc_b = pl.pallas_call(
    _tc_b_body,
    out_shape=jax.ShapeDtypeStruct((NPAD, H), jnp.float32),
)

_tc_c = pl.pallas_call(
    _tc_c_body,
    out_shape=jax.ShapeDtypeStruct((G, C), jnp.float32),
)


def kernel(x, edge_index, batch_num_nodes, W0, b0, W1, b1, Wlin, blin):
    src = edge_index[0]
    dst = edge_index[1]

    pad = N + jnp.arange(EPAD - E, dtype=jnp.int32) % PAD_SPREAD
    src_p = jnp.concatenate([src, pad]).reshape(NW, MSG_STEPS, CH)
    dst_p = jnp.concatenate([dst, pad]).reshape(NW, MSG_STEPS, CH)

    degp = _deg_kernel(src_p, dst_p)

    y0s = _tc_a(x, W0, degp)
    p1 = _msg_kernel(src_p, dst_p, y0s)
    z = _tc_b(p1, degp, b0.reshape(1, H), W1)
    p2 = _msg_kernel(src_p, dst_p, z)

    offsets = jnp.concatenate([
        jnp.zeros((1,), jnp.int32),
        jnp.cumsum(batch_num_nodes)[:-1].astype(jnp.int32),
    ])
    p10 = p2[:, offsets]
    din10 = (degp[0, 1] + degp[1, 1])[offsets].reshape(G, 1)
    return _tc_c(p10, din10, b1.reshape(1, H), Wlin, blin.reshape(1, C))


# speedup vs baseline: 1.1076x; 1.1076x over previous
import functools

import jax
import jax.numpy as jnp
from jax import lax
from jax.experimental import pallas as pl
from jax.experimental.pallas import tpu as pltpu
from jax.experimental.pallas import tpu_sc as plsc

N = 10000
E = 320000
D = 128
H = 64
C = 40
G = 10

NC, NS = 2, 16
NW = NC * NS
CH = 128

NPAD = 10240
ROWS_PER_TILE = NPAD // NS
PAD_SPREAD = NPAD - N

NBUF = 5
MSG_STEPS = 80
EPT = MSG_STEPS * CH
EPAD = EPT * NW
STEADY = MSG_STEPS - NBUF

DEG_SLICE = NPAD // NS

_MESH = plsc.VectorSubcoreMesh(core_axis_name="c", subcore_axis_name="s")
_SC_PARAMS = pltpu.CompilerParams(use_tc_tiling_on_sc=False)


@functools.partial(
    pl.kernel,
    out_type=jax.ShapeDtypeStruct((NC, 2, NPAD), jnp.float32),
    mesh=_MESH,
    scratch_types=[
        pltpu.VMEM((MSG_STEPS, CH), jnp.int32),
        pltpu.VMEM((MSG_STEPS, CH), jnp.int32),
        pltpu.VMEM((CH,), jnp.float32),
        pltpu.VMEM((DEG_SLICE,), jnp.float32),
        pltpu.VMEM_SHARED((NPAD,), jnp.float32),
        pltpu.VMEM_SHARED((NPAD,), jnp.float32),
    ],
    compiler_params=_SC_PARAMS,
)
def _deg_kernel(src_hbm, dst_hbm, out_hbm, srcs, dsts, ones_v, zer_v,
                acc_s, acc_d):
    cid = lax.axis_index("c")
    sid = lax.axis_index("s")
    wid = cid * NS + sid

    @pl.loop(0, DEG_SLICE // 16)
    def _zero(i):
        zer_v[pl.ds(i * 16, 16)] = jnp.zeros((16,), jnp.float32)

    @pl.loop(0, CH // 16)
    def _ones(i):
        ones_v[pl.ds(i * 16, 16)] = jnp.ones((16,), jnp.float32)

    pltpu.sync_copy(zer_v, acc_s.at[pl.ds(sid * DEG_SLICE, DEG_SLICE)])
    pltpu.sync_copy(zer_v, acc_d.at[pl.ds(sid * DEG_SLICE, DEG_SLICE)])
    pltpu.sync_copy(src_hbm.at[wid], srcs)
    pltpu.sync_copy(dst_hbm.at[wid], dsts)
    plsc.subcore_barrier()

    @pl.loop(0, MSG_STEPS)
    def _step(j):
        pltpu.sync_copy(ones_v, acc_s.at[srcs.at[j]], add=True)
        pltpu.sync_copy(ones_v, acc_d.at[dsts.at[j]], add=True)

    plsc.subcore_barrier()
    pltpu.sync_copy(acc_s.at[pl.ds(sid * DEG_SLICE, DEG_SLICE)],
                    out_hbm.at[cid, 0, pl.ds(sid * DEG_SLICE, DEG_SLICE)])
    pltpu.sync_copy(acc_d.at[pl.ds(sid * DEG_SLICE, DEG_SLICE)],
                    out_hbm.at[cid, 1, pl.ds(sid * DEG_SLICE, DEG_SLICE)])


_MSG_SCRATCH = (
    [pltpu.VMEM((MSG_STEPS, CH), jnp.int32)] * 2 +
    [pltpu.VMEM((CH, H), jnp.float32)] * (NBUF + 1) +
    [pltpu.SemaphoreType.DMA] * NBUF +
    [pltpu.VMEM_SHARED((NPAD, H), jnp.float32)]
)


@functools.partial(
    pl.kernel,
    out_type=jax.ShapeDtypeStruct((NC, NPAD, H), jnp.float32),
    mesh=_MESH,
    scratch_types=_MSG_SCRATCH,
    compiler_params=_SC_PARAMS,
)
def _msg_kernel(src_hbm, dst_hbm, feat_hbm, out_hbm, srcs, dsts, *rest):
    rowbufs = rest[:NBUF]
    zbuf = rest[NBUF]
    sems = rest[NBUF + 1:2 * NBUF + 1]
    acc = rest[2 * NBUF + 1]
    cid = lax.axis_index("c")
    sid = lax.axis_index("s")
    wid = cid * NS + sid

    @pl.loop(0, CH)
    def _zrow(i):
        @pl.loop(0, H // 16)
        def _zcol(j):
            zbuf[i, pl.ds(j * 16, 16)] = jnp.zeros((16,), jnp.float32)

    for i in range(ROWS_PER_TILE // CH):
        pltpu.sync_copy(zbuf, acc.at[pl.ds(sid * ROWS_PER_TILE + i * CH, CH)])
    plsc.subcore_barrier()

    pltpu.sync_copy(src_hbm.at[wid], srcs)
    pltpu.sync_copy(dst_hbm.at[wid], dsts)

    for b in range(NBUF):
        pltpu.async_copy(feat_hbm.at[srcs.at[b]], rowbufs[b], sems[b])

    @pl.loop(0, STEADY, step=NBUF)
    def _step(j):
        for b in range(NBUF):
            rows, sem = rowbufs[b], sems[b]
            pltpu.make_async_copy(feat_hbm.at[srcs.at[j + b]], rows,
                                  sem).wait()
            pltpu.sync_copy(rows, acc.at[dsts.at[j + b]], add=True)
            pltpu.async_copy(feat_hbm.at[srcs.at[j + b + NBUF]], rows, sem)

    for b in range(NBUF):
        rows, sem = rowbufs[b], sems[b]
        c = STEADY + b
        pltpu.make_async_copy(feat_hbm.at[srcs.at[c]], rows, sem).wait()
        pltpu.sync_copy(rows, acc.at[dsts.at[c]], add=True)

    plsc.subcore_barrier()
    for i in range(ROWS_PER_TILE // CH):
        r0 = sid * ROWS_PER_TILE + i * CH
        pltpu.sync_copy(acc.at[pl.ds(r0, CH)], out_hbm.at[cid, pl.ds(r0, CH)])


_PREC = lax.Precision.HIGHEST


def _gelu(x):
    return 0.5 * x * (1.0 + lax.erf(x * 0.7071067811865476))


def _rdeg(degp_ref, which):
    d = degp_ref[0, which, :N] + degp_ref[1, which, :N]
    return lax.rsqrt(jnp.maximum(d, 1.0)).reshape(N, 1)


def _tc_a_body(x_ref, w0_ref, degp_ref, y_ref):
    y0 = jnp.dot(x_ref[...], w0_ref[...], precision=_PREC)
    y_ref[:N] = y0 * _rdeg(degp_ref, 0)
    y_ref[N:] = jnp.zeros((NPAD - N, H), jnp.float32)


def _tc_b_body(p_ref, degp_ref, b0_ref, w1_ref, z_ref):
    m = p_ref[0, :N] + p_ref[1, :N]
    h = _gelu(m * _rdeg(degp_ref, 1) + b0_ref[...])
    z_ref[:N] = jnp.dot(h, w1_ref[...], precision=_PREC) * _rdeg(degp_ref, 0)
    z_ref[N:] = jnp.zeros((NPAD - N, H), jnp.float32)


def _tc_c_body(off_ref, p2_ref, din_ref, b1_ref, wl_ref, bl_ref, o_ref,
               buf, sem):
    dmas = []
    for c in range(NC):
        for g in range(G):
            dma = pltpu.make_async_copy(
                p2_ref.at[c, pl.ds(off_ref[g], 1)],
                buf.at[c, pl.ds(g, 1)], sem)
            dma.start()
            dmas.append(dma)
    for dma in dmas:
        dma.wait()
    m = buf[0] + buf[1]
    rin = lax.rsqrt(jnp.maximum(din_ref[...], 1.0))
    h = _gelu(m * rin + b1_ref[...])
    o_ref[...] = jnp.dot(h, wl_ref[...], precision=_PREC) + bl_ref[...]


_tc_a = pl.pallas_call(
    _tc_a_body,
    out_shape=jax.ShapeDtypeStruct((NPAD, H), jnp.float32),
)

_tc_b = pl.pallas_call(
    _tc_b_body,
    out_shape=jax.ShapeDtypeStruct((NPAD, H), jnp.float32),
)

_tc_c = pl.pallas_call(
    _tc_c_body,
    out_shape=jax.ShapeDtypeStruct((G, C), jnp.float32),
    in_specs=[
        pl.BlockSpec(memory_space=pltpu.MemorySpace.SMEM),
        pl.BlockSpec(memory_space=pltpu.MemorySpace.HBM),
        pl.BlockSpec(memory_space=pltpu.VMEM),
        pl.BlockSpec(memory_space=pltpu.VMEM),
        pl.BlockSpec(memory_space=pltpu.VMEM),
        pl.BlockSpec(memory_space=pltpu.VMEM),
    ],
    scratch_shapes=[
        pltpu.VMEM((NC, G, H), jnp.float32),
        pltpu.SemaphoreType.DMA,
    ],
)


def kernel(x, edge_index, batch_num_nodes, W0, b0, W1, b1, Wlin, blin):
    src = edge_index[0]
    dst = edge_index[1]

    pad = N + jnp.arange(EPAD - E, dtype=jnp.int32) % PAD_SPREAD
    src_p = jnp.concatenate([src, pad]).reshape(NW, MSG_STEPS, CH)
    dst_p = jnp.concatenate([dst, pad]).reshape(NW, MSG_STEPS, CH)

    degp = _deg_kernel(src_p, dst_p)

    y0s = _tc_a(x, W0, degp)
    p1 = _msg_kernel(src_p, dst_p, y0s)
    z = _tc_b(p1, degp, b0.reshape(1, H), W1)
    p2 = _msg_kernel(src_p, dst_p, z)

    offsets = jnp.concatenate([
        jnp.zeros((1,), jnp.int32),
        jnp.cumsum(batch_num_nodes)[:-1].astype(jnp.int32),
    ])
    din10 = (degp[0, 1] + degp[1, 1])[offsets].reshape(G, 1)
    return _tc_c(offsets, p2, din10, b1.reshape(1, H), Wlin,
                 blin.reshape(1, C))
